# baseline (speedup 1.0000x reference)
import jax
import jax.numpy as jnp
from jax import lax
from jax.experimental import pallas as pl
from jax.experimental.pallas import tpu as pltpu
from jax.experimental.pallas import tpu_sc as plsc

ENT_NUM = 1000000
REL_NUM = 1000
DIM = 64
B = 4096
NEG = 200

_INFO = plsc.get_sparse_core_info()
NC = _INFO.num_cores
NS = _INFO.num_subcores
NW = NC * NS
CH = 128
NBB = B // CH
DB = DIM // 8
NTR = NEG // 8
CHP = CH + 1


def _transpose_chunk(rows_v, trans_v, p):
    lane = lax.iota(jnp.int32, 16)
    idb = lane // 8
    ids = lax.rem(lane, 8)
    pb = jnp.full((16,), p, jnp.int32)

    def bq_body(bq, carry):
        vs = []
        for j in range(8):
            b = bq * 8 + j
            for dq in range(DIM // 16):
                vs.append((rows_v[p, b, pl.ds(dq * 16, 16)],
                           idb + dq * 2, jnp.full((16,), b, jnp.int32)))
        for v, dtile, bs in vs:
            plsc.store_scatter(trans_v, [pb, dtile, ids, bs], v)
        return carry

    lax.fori_loop(0, CH // 8, bq_body, 0)


def _sc_gather(ent_hbm, rel_hbm, negidx_hbm, posidx_hbm,
               neg_out, h_out, r_out, t_out,
               idx_v, rows_v, trans_v, gsem, wsem):
    wid = lax.axis_index("s") * NC + lax.axis_index("c")

    for t in range(NTR):
        pltpu.sync_copy(negidx_hbm.at[t, wid], idx_v.at[pl.ds(t * 8, 8)])

    def fire_gather(n, p):
        pltpu.async_copy(ent_hbm.at[idx_v.at[n]], rows_v.at[p], gsem)

    def drain_gather(p):
        pltpu.make_async_copy(ent_hbm.at[pl.ds(0, CH)],
                              rows_v.at[p], gsem).wait()

    def wb_src(p):
        return trans_v.at[p, :, :, pl.ds(0, CH)]

    def drain_wb(p):
        pltpu.make_async_copy(neg_out.at[0, :, 0], wb_src(p), wsem).wait()

    fire_gather(0, 0)

    def chunk_body(n, carry):
        p = lax.rem(n, 2)
        drain_gather(p)

        @pl.when(n + 1 < NEG)
        def _():
            fire_gather(n + 1, 1 - p)

        @pl.when(n >= 2)
        def _():
            drain_wb(p)
        _transpose_chunk(rows_v, trans_v, p)
        pltpu.async_copy(wb_src(p), neg_out.at[n, :, wid], wsem)
        return carry

    lax.fori_loop(0, NEG, chunk_body, 0)
    drain_wb(lax.rem(NEG, 2))
    drain_wb(lax.rem(NEG + 1, 2))

    for which, (table, out) in enumerate(
            [(ent_hbm, h_out), (rel_hbm, r_out), (ent_hbm, t_out)]):
        r = which * NBB + wid
        pltpu.sync_copy(posidx_hbm.at[r // 8], idx_v.at[pl.ds(0, 8)])
        pltpu.async_copy(table.at[idx_v.at[lax.rem(r, 8)]],
                         rows_v.at[0], gsem).wait()
        _transpose_chunk(rows_v, trans_v, 0)
        pltpu.sync_copy(wb_src(0), out.at[:, wid])


@jax.jit
def _run(pos_sample, neg_sample, ent_embd, rel_embd):
    entL = ent_embd
    negidx4 = neg_sample.T.reshape(NTR, 8, NBB, CH).transpose(0, 2, 1, 3)
    posidx = pos_sample.T.reshape(3 * B // (8 * CH), 8, CH)

    mesh = plsc.VectorSubcoreMesh(core_axis_name="c", subcore_axis_name="s")
    small = jax.ShapeDtypeStruct((DB, NBB, 8, CH), jnp.float32)
    neg5, h5, r5, t5 = pl.kernel(
        _sc_gather,
        out_type=[
            jax.ShapeDtypeStruct((NEG, DB, NBB, 8, CH), jnp.float32),
            small, small, small,
        ],
        mesh=mesh,
        scratch_types=[
            pltpu.VMEM((NEG, CH), jnp.int32),
            pltpu.VMEM((2, CH, DIM), jnp.float32),
            pltpu.VMEM((2, DB, 8, CHP), jnp.float32),
            pltpu.SemaphoreType.DMA,
            pltpu.SemaphoreType.DMA,
        ],
        compiler_params=pltpu.CompilerParams(use_tc_tiling_on_sc=False,
                                             needs_layout_passes=False),
        name="kge_embed_gather",
    )(entL, rel_embd, negidx4, posidx)

    neg = neg5.transpose(2, 4, 0, 1, 3).reshape(B, NEG, DIM)
    head = h5.transpose(1, 3, 0, 2).reshape(B, 1, DIM)
    relation = r5.transpose(1, 3, 0, 2).reshape(B, 1, DIM)
    tail = t5.transpose(1, 3, 0, 2).reshape(B, 1, DIM)
    return head, relation, tail, neg


def kernel(pos_sample, neg_sample, ent_embd, rel_embd):
    return _run(pos_sample, neg_sample, ent_embd, rel_embd)


# speedup vs baseline: 1.0080x; 1.0080x over previous
import jax
import jax.numpy as jnp
from jax import lax
from jax.experimental import pallas as pl
from jax.experimental.pallas import tpu as pltpu
from jax.experimental.pallas import tpu_sc as plsc

ENT_NUM = 1000000
REL_NUM = 1000
DIM = 64
B = 4096
NEG = 200

_INFO = plsc.get_sparse_core_info()
NC = _INFO.num_cores
NS = _INFO.num_subcores
NW = NC * NS
CH = 128
NBB = B // CH
DB = DIM // 8
NTR = NEG // 8
CHP = CH + 1


def _transpose_chunk(rows_v, trans_v, p, pt):
    lane = lax.iota(jnp.int32, 16)
    idb = lane // 8
    ids = lax.rem(lane, 8)
    pb = jnp.full((16,), pt, jnp.int32)

    def bq_body(bq, carry):
        vs = []
        for j in range(4):
            b = bq * 4 + j
            for dq in range(DIM // 16):
                vs.append((rows_v[p, b, pl.ds(dq * 16, 16)],
                           idb + dq * 2, jnp.full((16,), b, jnp.int32)))
        for v, dtile, bs in vs:
            plsc.store_scatter(trans_v, [pb, dtile, ids, bs], v)
        return carry

    lax.fori_loop(0, CH // 4, bq_body, 0)


def _sc_gather(ent_hbm, rel_hbm, negidx_hbm, posidx_hbm,
               neg_out, h_out, r_out, t_out,
               idx_v, rows_v, trans_v, gsem, wsem):
    wid = lax.axis_index("s") * NC + lax.axis_index("c")

    for t in range(NTR):
        pltpu.sync_copy(negidx_hbm.at[t, wid], idx_v.at[pl.ds(t * 8, 8)])

    def fire_gather(n, p):
        pltpu.async_copy(ent_hbm.at[idx_v.at[n]], rows_v.at[p], gsem)

    def drain_gather(p):
        pltpu.make_async_copy(ent_hbm.at[pl.ds(0, CH)],
                              rows_v.at[p], gsem).wait()

    def wb_src(p):
        return trans_v.at[p, :, :, pl.ds(0, CH)]

    def drain_wb(p):
        pltpu.make_async_copy(neg_out.at[0, :, 0], wb_src(p), wsem).wait()

    fire_gather(0, 0)
    fire_gather(1, 1)

    def chunk_body(n, carry):
        p = lax.rem(n, 3)
        pt = lax.rem(n, 2)
        drain_gather(p)

        @pl.when(n + 2 < NEG)
        def _():
            fire_gather(n + 2, lax.rem(n + 2, 3))

        @pl.when(n >= 2)
        def _():
            drain_wb(pt)
        _transpose_chunk(rows_v, trans_v, p, pt)
        pltpu.async_copy(wb_src(pt), neg_out.at[n, :, wid], wsem)
        return carry

    lax.fori_loop(0, NEG, chunk_body, 0)
    drain_wb(lax.rem(NEG, 2))
    drain_wb(lax.rem(NEG + 1, 2))

    for which, (table, out) in enumerate(
            [(ent_hbm, h_out), (rel_hbm, r_out), (ent_hbm, t_out)]):
        r = which * NBB + wid
        pltpu.sync_copy(posidx_hbm.at[r // 8], idx_v.at[pl.ds(0, 8)])
        pltpu.async_copy(table.at[idx_v.at[lax.rem(r, 8)]],
                         rows_v.at[0], gsem).wait()
        _transpose_chunk(rows_v, trans_v, 0, 0)
        pltpu.sync_copy(wb_src(0), out.at[:, wid])


@jax.jit
def _run(pos_sample, neg_sample, ent_embd, rel_embd):
    entL = ent_embd
    negidx4 = neg_sample.T.reshape(NTR, 8, NBB, CH).transpose(0, 2, 1, 3)
    posidx = pos_sample.T.reshape(3 * B // (8 * CH), 8, CH)

    mesh = plsc.VectorSubcoreMesh(core_axis_name="c", subcore_axis_name="s")
    small = jax.ShapeDtypeStruct((DB, NBB, 8, CH), jnp.float32)
    neg5, h5, r5, t5 = pl.kernel(
        _sc_gather,
        out_type=[
            jax.ShapeDtypeStruct((NEG, DB, NBB, 8, CH), jnp.float32),
            small, small, small,
        ],
        mesh=mesh,
        scratch_types=[
            pltpu.VMEM((NEG, CH), jnp.int32),
            pltpu.VMEM((3, CH, DIM), jnp.float32),
            pltpu.VMEM((2, DB, 8, CHP), jnp.float32),
            pltpu.SemaphoreType.DMA,
            pltpu.SemaphoreType.DMA,
        ],
        compiler_params=pltpu.CompilerParams(use_tc_tiling_on_sc=False,
                                             needs_layout_passes=False),
        name="kge_embed_gather",
    )(entL, rel_embd, negidx4, posidx)

    neg = neg5.transpose(2, 4, 0, 1, 3).reshape(B, NEG, DIM)
    head = h5.transpose(1, 3, 0, 2).reshape(B, 1, DIM)
    relation = r5.transpose(1, 3, 0, 2).reshape(B, 1, DIM)
    tail = t5.transpose(1, 3, 0, 2).reshape(B, 1, DIM)
    return head, relation, tail, neg


def kernel(pos_sample, neg_sample, ent_embd, rel_embd):
    return _run(pos_sample, neg_sample, ent_embd, rel_embd)
